# baseline (speedup 1.0000x reference)
import functools

import jax
import jax.numpy as jnp
from jax import lax
from jax.experimental import pallas as pl
from jax.experimental.pallas import tpu as pltpu

T = 2048
K = 2
E = 8
H = 1024
I = 2048

BM = 256
NB = (T * K) // BM + E
P = NB * BM
BI = 512
NI = I // BI


def _route_metadata(top_k_index, top_k_weights):
    tk = top_k_index.reshape(-1).astype(jnp.int32)
    wf = top_k_weights.reshape(-1).astype(jnp.float32)
    onehot = (tk[:, None] == jnp.arange(E, dtype=jnp.int32)[None, :]).astype(
        jnp.int32)
    csum = jnp.cumsum(onehot, axis=0)
    rank = jnp.take_along_axis(csum, tk[:, None], axis=1)[:, 0] - 1
    g = csum[-1]
    nbe = (g + BM - 1) // BM
    cum_nb = jnp.cumsum(nbe).astype(jnp.int32)
    row_start = (cum_nb - nbe) * BM
    ppos = row_start[tk] + rank
    src_row = jnp.zeros((P,), jnp.int32).at[ppos].set(
        jnp.arange(T * K, dtype=jnp.int32) // K)
    w_pad = jnp.zeros((P,), jnp.float32).at[ppos].set(wf)
    b_used = cum_nb[-1]
    bidx = jnp.minimum(jnp.arange(NB, dtype=jnp.int32), b_used - 1)
    block_expert = jnp.searchsorted(cum_nb, bidx, side='right').astype(jnp.int32)
    be_arr = jnp.concatenate([block_expert, b_used[None]])
    pos = ppos.reshape(T, K)
    return src_row, w_pad, be_arr, pos[:, 0], pos[:, 1]


def _gmm_body(be_ref, x_ref, wg_ref, wu_ref, wd_ref, w_ref, o_ref):
    b = pl.program_id(0)

    @pl.when(b < be_ref[NB])
    def _():
        x = x_ref[...].astype(jnp.bfloat16)
        w = w_ref[0, 0][:, None]
        for i in range(NI):
            cols = pl.ds(i * BI, BI)
            wg = wg_ref[0, :, cols].astype(jnp.bfloat16)
            wu = wu_ref[0, :, cols].astype(jnp.bfloat16)
            wd = wd_ref[0, cols, :].astype(jnp.bfloat16)
            gate = jnp.dot(x, wg, preferred_element_type=jnp.float32)
            up = jnp.dot(x, wu, preferred_element_type=jnp.float32)
            h = (gate * jax.nn.sigmoid(gate)) * up * w
            part = jnp.dot(h.astype(jnp.bfloat16), wd,
                           preferred_element_type=jnp.float32)
            if i == 0:
                o_ref[...] = part
            else:
                o_ref[...] += part


def _grouped_mlp(block_expert, x_pad, Wg, Wu, Wd, w_pad3):
    grid_spec = pltpu.PrefetchScalarGridSpec(
        num_scalar_prefetch=1,
        grid=(NB,),
        in_specs=[
            pl.BlockSpec((BM, H), lambda b, be: (b, 0)),
            pl.BlockSpec((1, H, I), lambda b, be: (be[b], 0, 0)),
            pl.BlockSpec((1, H, I), lambda b, be: (be[b], 0, 0)),
            pl.BlockSpec((1, I, H), lambda b, be: (be[b], 0, 0)),
            pl.BlockSpec((1, 1, BM), lambda b, be: (b, 0, 0)),
        ],
        out_specs=pl.BlockSpec((BM, H), lambda b, be: (b, 0)),
    )
    return pl.pallas_call(
        _gmm_body,
        grid_spec=grid_spec,
        out_shape=jax.ShapeDtypeStruct((P, H), jnp.float32),
        compiler_params=pltpu.CompilerParams(
            dimension_semantics=("arbitrary",),
            vmem_limit_bytes=110 * 1024 * 1024),
    )(block_expert, x_pad, Wg, Wu, Wd, w_pad3)


def kernel(hidden_states, top_k_index, top_k_weights, Wg, Wu, Wd):
    src_row, w_pad, be_arr, pos0, pos1 = _route_metadata(
        top_k_index, top_k_weights)
    x_pad = hidden_states[src_row]
    y_pad = _grouped_mlp(be_arr, x_pad, Wg, Wu, Wd,
                         w_pad.reshape(NB, 1, BM))
    return y_pad[pos0] + y_pad[pos1]


# speedup vs baseline: 1.1468x; 1.1468x over previous
import functools

import jax
import jax.numpy as jnp
from jax import lax
from jax.experimental import pallas as pl
from jax.experimental.pallas import tpu as pltpu

T = 2048
K = 2
E = 8
H = 1024
I = 2048

BM = 256
NB = (T * K) // BM + E
P = NB * BM
BI = 512
NI = I // BI

G = T * K
GC = 32
GR = G // GC
ER = E * GC


def _meta_body(tk_ref, ppos_ref, binfo_ref):
    f32 = jnp.float32
    tk = tk_ref[...]
    tk8 = jnp.tile(tk, (E, 1))
    erow = lax.broadcasted_iota(jnp.int32, (ER, GR), 0) // GC
    oh = (tk8 == erow).astype(f32)
    ii = lax.broadcasted_iota(jnp.int32, (GR, GR), 0)
    jj = lax.broadcasted_iota(jnp.int32, (GR, GR), 1)
    U = (ii <= jj).astype(f32)
    csum = jnp.dot(oh, U, preferred_element_type=f32)
    tot = csum[:, GR - 1:GR]
    ri = lax.broadcasted_iota(jnp.int32, (ER, ER), 0)
    ci = lax.broadcasted_iota(jnp.int32, (ER, ER), 1)
    same_e = (ri // GC) == (ci // GC)
    Mx = (same_e & ((ci % GC) < (ri % GC))).astype(f32)
    Me = same_e.astype(f32)
    rep = (ci % GC) == 0
    Mex = (((ci // GC) < (ri // GC)) & rep).astype(f32)
    Min = (((ci // GC) <= (ri // GC)) & rep).astype(f32)
    excl = jnp.dot(Mx, tot, preferred_element_type=f32)
    gvec = jnp.dot(Me, tot, preferred_element_type=f32)
    nbe = jnp.floor((gvec + (BM - 1)) * (1.0 / BM))
    rs = jnp.dot(Mex, nbe, preferred_element_type=f32) * BM
    cn = jnp.dot(Min, nbe, preferred_element_type=f32)
    pall = csum - 1.0 + excl + rs
    ppos = jnp.sum((oh * pall).reshape(E, GC, GR), axis=0)
    ppos_ref[...] = ppos.astype(jnp.int32)
    bu = jnp.max(cn, keepdims=True)
    bidx = jnp.minimum(
        lax.broadcasted_iota(jnp.int32, (1, GR), 1).astype(f32),
        bu - 1.0)
    rmask = (lax.broadcasted_iota(jnp.int32, (ER, 1), 0) % GC
             == 0).astype(f32)
    be = jnp.sum((cn <= bidx).astype(f32) * rmask, axis=0,
                 keepdims=True)
    binfo_ref[0:1, :] = be.astype(jnp.int32)
    binfo_ref[1:2, :] = jnp.broadcast_to(bu, (1, GR)).astype(jnp.int32)


def _route_metadata(top_k_index):
    tk_cr = top_k_index.reshape(GC, GR).astype(jnp.int32)
    ppos_cr, binfo = pl.pallas_call(
        _meta_body,
        out_shape=(jax.ShapeDtypeStruct((GC, GR), jnp.int32),
                   jax.ShapeDtypeStruct((8, GR), jnp.int32)),
    )(tk_cr)
    ppos = ppos_cr.reshape(G)
    be_arr = jnp.concatenate([binfo[0, :NB], binfo[1, :1]])
    pos = ppos.reshape(T, K)
    return be_arr, ppos, pos[:, 0], pos[:, 1]


def _gmm_body(be_ref, x_ref, wg_ref, wu_ref, wd_ref, o_ref):
    b = pl.program_id(0)

    @pl.when(b < be_ref[NB])
    def _():
        x = x_ref[...].astype(jnp.bfloat16)
        for i in range(NI):
            cols = pl.ds(i * BI, BI)
            wg = wg_ref[0, :, cols].astype(jnp.bfloat16)
            wu = wu_ref[0, :, cols].astype(jnp.bfloat16)
            wd = wd_ref[0, cols, :].astype(jnp.bfloat16)
            gate = jnp.dot(x, wg, preferred_element_type=jnp.float32)
            up = jnp.dot(x, wu, preferred_element_type=jnp.float32)
            h = (gate * jax.nn.sigmoid(gate)) * up
            part = jnp.dot(h.astype(jnp.bfloat16), wd,
                           preferred_element_type=jnp.float32)
            if i == 0:
                o_ref[...] = part
            else:
                o_ref[...] += part


def _grouped_mlp(be_arr, x_pad, Wg, Wu, Wd):
    grid_spec = pltpu.PrefetchScalarGridSpec(
        num_scalar_prefetch=1,
        grid=(NB,),
        in_specs=[
            pl.BlockSpec((BM, H), lambda b, be: (b, 0)),
            pl.BlockSpec((1, H, I), lambda b, be: (be[b], 0, 0)),
            pl.BlockSpec((1, H, I), lambda b, be: (be[b], 0, 0)),
            pl.BlockSpec((1, I, H), lambda b, be: (be[b], 0, 0)),
        ],
        out_specs=pl.BlockSpec((BM, H), lambda b, be: (b, 0)),
    )
    return pl.pallas_call(
        _gmm_body,
        grid_spec=grid_spec,
        out_shape=jax.ShapeDtypeStruct((P, H), jnp.float32),
        compiler_params=pltpu.CompilerParams(
            dimension_semantics=("arbitrary",),
            vmem_limit_bytes=60 * 1024 * 1024),
    )(be_arr, x_pad, Wg, Wu, Wd)


def kernel(hidden_states, top_k_index, top_k_weights, Wg, Wu, Wd):
    be_arr, ppos, pos0, pos1 = _route_metadata(top_k_index)
    src_row = jnp.zeros((P,), jnp.int32).at[ppos].set(
        jnp.arange(G, dtype=jnp.int32) // K)
    x_pad = hidden_states[src_row]
    y_pad = _grouped_mlp(be_arr, x_pad, Wg, Wu, Wd)
    w0 = top_k_weights[:, 0:1]
    w1 = top_k_weights[:, 1:2]
    return w0 * y_pad[pos0] + w1 * y_pad[pos1]


# speedup vs baseline: 1.3944x; 1.2159x over previous
import functools

import jax
import jax.numpy as jnp
from jax import lax
from jax.experimental import pallas as pl
from jax.experimental.pallas import tpu as pltpu
from jax.experimental.pallas import tpu_sc as plsc

_SC_INFO = plsc.get_sparse_core_info()
NC = _SC_INFO.num_cores
NS = _SC_INFO.num_subcores
NW = NC * NS

T = 2048
K = 2
E = 8
H = 1024
I = 2048

BM = 256
NB = (T * K) // BM + E
P = NB * BM
BI = 512
NI = I // BI

G = T * K
GC = 32
GR = G // GC
ER = E * GC


def _meta_body(tk_ref, ppos_ref, binfo_ref):
    f32 = jnp.float32
    tk = tk_ref[...]
    tk8 = jnp.tile(tk, (E, 1))
    erow = lax.broadcasted_iota(jnp.int32, (ER, GR), 0) // GC
    oh = (tk8 == erow).astype(f32)
    ii = lax.broadcasted_iota(jnp.int32, (GR, GR), 0)
    jj = lax.broadcasted_iota(jnp.int32, (GR, GR), 1)
    U = (ii <= jj).astype(f32)
    csum = jnp.dot(oh, U, preferred_element_type=f32)
    tot = csum[:, GR - 1:GR]
    ri = lax.broadcasted_iota(jnp.int32, (ER, ER), 0)
    ci = lax.broadcasted_iota(jnp.int32, (ER, ER), 1)
    same_e = (ri // GC) == (ci // GC)
    Mx = (same_e & ((ci % GC) < (ri % GC))).astype(f32)
    Me = same_e.astype(f32)
    rep = (ci % GC) == 0
    Mex = (((ci // GC) < (ri // GC)) & rep).astype(f32)
    Min = (((ci // GC) <= (ri // GC)) & rep).astype(f32)
    excl = jnp.dot(Mx, tot, preferred_element_type=f32)
    gvec = jnp.dot(Me, tot, preferred_element_type=f32)
    nbe = jnp.floor((gvec + (BM - 1)) * (1.0 / BM))
    rs = jnp.dot(Mex, nbe, preferred_element_type=f32) * BM
    cn = jnp.dot(Min, nbe, preferred_element_type=f32)
    pall = csum - 1.0 + excl + rs
    ppos = jnp.sum((oh * pall).reshape(E, GC, GR), axis=0)
    ppos_ref[...] = ppos.astype(jnp.int32)
    bu = jnp.max(cn, keepdims=True)
    bidx = jnp.minimum(
        lax.broadcasted_iota(jnp.int32, (1, GR), 1).astype(f32),
        bu - 1.0)
    rmask = (lax.broadcasted_iota(jnp.int32, (ER, 1), 0) % GC
             == 0).astype(f32)
    be = jnp.sum((cn <= bidx).astype(f32) * rmask, axis=0,
                 keepdims=True)
    binfo_ref[0:1, :] = be.astype(jnp.int32)
    binfo_ref[1:2, :] = jnp.broadcast_to(bu, (1, GR)).astype(jnp.int32)


def _route_metadata(top_k_index):
    tk_cr = top_k_index.reshape(GC, GR).astype(jnp.int32)
    ppos_cr, binfo = pl.pallas_call(
        _meta_body,
        out_shape=(jax.ShapeDtypeStruct((GC, GR), jnp.int32),
                   jax.ShapeDtypeStruct((8, GR), jnp.int32)),
    )(tk_cr)
    ppos = ppos_cr.reshape(G)
    be_arr = jnp.concatenate([binfo[0, :NB], binfo[1, :1]])
    pos = ppos.reshape(T, K)
    return be_arr, ppos, pos[:, 0], pos[:, 1]


_TPW = T // NW


def _dispatch_body(hid_hbm, p0_hbm, p1_hbm, out_hbm, rows_v, i0_v, i1_v, sem):
    wid = lax.axis_index("s") * NC + lax.axis_index("c")
    base = wid * _TPW
    pltpu.sync_copy(hid_hbm.at[pl.ds(base, _TPW)], rows_v)
    pltpu.sync_copy(p0_hbm.at[pl.ds(base, _TPW)], i0_v)
    pltpu.sync_copy(p1_hbm.at[pl.ds(base, _TPW)], i1_v)
    pltpu.async_copy(rows_v, out_hbm.at[i0_v], sem).wait()
    pltpu.async_copy(rows_v, out_hbm.at[i1_v], sem).wait()


def _sc_dispatch(hidden_states, pos0, pos1):
    mesh = plsc.VectorSubcoreMesh(core_axis_name="c", subcore_axis_name="s")
    return pl.kernel(
        _dispatch_body,
        out_type=jax.ShapeDtypeStruct((P, H), jnp.float32),
        mesh=mesh,
        scratch_types=[
            pltpu.VMEM((_TPW, H), jnp.float32),
            pltpu.VMEM((_TPW,), jnp.int32),
            pltpu.VMEM((_TPW,), jnp.int32),
            pltpu.SemaphoreType.DMA,
        ],
    )(hidden_states, pos0, pos1)


def _gmm_body(be_ref, x_ref, wg_ref, wu_ref, wd_ref, o_ref):
    b = pl.program_id(0)

    @pl.when(b < be_ref[NB])
    def _():
        x = x_ref[...].astype(jnp.bfloat16)
        for i in range(NI):
            cols = pl.ds(i * BI, BI)
            wg = wg_ref[0, :, cols].astype(jnp.bfloat16)
            wu = wu_ref[0, :, cols].astype(jnp.bfloat16)
            wd = wd_ref[0, cols, :].astype(jnp.bfloat16)
            gate = jnp.dot(x, wg, preferred_element_type=jnp.float32)
            up = jnp.dot(x, wu, preferred_element_type=jnp.float32)
            h = (gate * jax.nn.sigmoid(gate)) * up
            part = jnp.dot(h.astype(jnp.bfloat16), wd,
                           preferred_element_type=jnp.float32)
            if i == 0:
                o_ref[...] = part
            else:
                o_ref[...] += part


def _grouped_mlp(be_arr, x_pad, Wg, Wu, Wd):
    grid_spec = pltpu.PrefetchScalarGridSpec(
        num_scalar_prefetch=1,
        grid=(NB,),
        in_specs=[
            pl.BlockSpec((BM, H), lambda b, be: (b, 0)),
            pl.BlockSpec((1, H, I), lambda b, be: (be[b], 0, 0)),
            pl.BlockSpec((1, H, I), lambda b, be: (be[b], 0, 0)),
            pl.BlockSpec((1, I, H), lambda b, be: (be[b], 0, 0)),
        ],
        out_specs=pl.BlockSpec((BM, H), lambda b, be: (b, 0)),
    )
    return pl.pallas_call(
        _gmm_body,
        grid_spec=grid_spec,
        out_shape=jax.ShapeDtypeStruct((P, H), jnp.float32),
        compiler_params=pltpu.CompilerParams(
            dimension_semantics=("arbitrary",),
            vmem_limit_bytes=60 * 1024 * 1024),
    )(be_arr, x_pad, Wg, Wu, Wd)


def kernel(hidden_states, top_k_index, top_k_weights, Wg, Wu, Wd):
    be_arr, ppos, pos0, pos1 = _route_metadata(top_k_index)
    x_pad = _sc_dispatch(hidden_states, pos0, pos1)
    y_pad = _grouped_mlp(be_arr, x_pad, Wg, Wu, Wd)
    w0 = top_k_weights[:, 0:1]
    w1 = top_k_weights[:, 1:2]
    return w0 * y_pad[pos0] + w1 * y_pad[pos1]
